# baseline (speedup 1.0000x reference)
import functools

import jax
import jax.numpy as jnp
from jax import lax
from jax.experimental import pallas as pl
from jax.experimental.pallas import tpu as pltpu
from jax.experimental.pallas import tpu_sc as plsc

def _gelu(t):
    return 0.5 * t * (1.0 + lax.erf(t * 0.7071067811865476))


N = 10000
E = 320000
W = 128
H = 64

NC = 2
NS = 16
NW = NC * NS
C = 5
EC = E // C
CH = 80
PW = EC // (NW * CH)
CHG = 40
K_FIRE = 10
PWG = EC // (NW * CHG)
OUTER = PWG // K_FIRE
GB = K_FIRE * CHG

def _mesh():
    return plsc.VectorSubcoreMesh(core_axis_name="c", subcore_axis_name="s",
                                  num_cores=NC, num_subcores=NS)


def _k1_body(x_ref, wpre_ref, bpre_ref, w3_ref, b3_ref, w0_ref, b0_ref,
             y3_ref, z_ref):
    xx = jnp.dot(x_ref[...], wpre_ref[...], preferred_element_type=jnp.float32)
    xx = xx + bpre_ref[...]
    m = jnp.mean(xx, axis=-1, keepdims=True)
    v = jnp.mean(jnp.square(xx - m), axis=-1, keepdims=True)
    xx = (xx - m) / jnp.sqrt(v + 1e-5)
    y3 = jnp.dot(xx, w3_ref[...], preferred_element_type=jnp.float32)
    y3_ref[...] = y3 + b3_ref[...]
    z = jnp.dot(xx, w0_ref[...], preferred_element_type=jnp.float32)
    z_ref[...] = _gelu(z + b0_ref[...])


def _k1(x, w_pre, b_pre, w3, b3, w0, b0):
    R = 1000
    full = pl.BlockSpec((W, W), lambda i: (0, 0))
    row = pl.BlockSpec((1, W), lambda i: (0, 0))
    return pl.pallas_call(
        _k1_body,
        grid=(N // R,),
        in_specs=[pl.BlockSpec((R, W), lambda i: (i, 0)),
                  full, row, full, row, full, row],
        out_specs=[pl.BlockSpec((R, W), lambda i: (i, 0)),
                   pl.BlockSpec((R, W), lambda i: (i, 0))],
        out_shape=[jax.ShapeDtypeStruct((N, W), jnp.float32),
                   jax.ShapeDtypeStruct((N, W), jnp.float32)],
    )(x, w_pre, b_pre, w3, b3, w0, b0)


def _s1_body(chunk, y3_hbm, src_hbm, g_hbm, idx_v, gbuf, sem0, sem1):
    cid = lax.axis_index("c")
    sid = lax.axis_index("s")
    wid = sid * NC + cid
    row0 = wid * PWG
    pltpu.sync_copy(src_hbm.at[chunk, wid], idx_v)
    sems = (sem0, sem1)

    def fire(j, b):
        for u in range(K_FIRE):
            pltpu.async_copy(y3_hbm.at[idx_v.at[j * K_FIRE + u]],
                             gbuf.at[b, pl.ds(u * CHG, CHG)], sems[b])

    def drain(b):
        for u in range(K_FIRE):
            pltpu.make_async_copy(y3_hbm.at[idx_v.at[0]],
                                  gbuf.at[b, pl.ds(u * CHG, CHG)],
                                  sems[b]).wait()

    def wout(j, b):
        eoff = (row0 + j * K_FIRE) * CHG
        pltpu.sync_copy(gbuf.at[b], g_hbm.at[pl.ds(eoff, GB)])

    fire(0, 0)

    def pair(p, carry):
        j0 = 2 * p
        drain(0)
        fire(j0 + 1, 1)
        wout(j0, 0)
        drain(1)
        fire(j0 + 2, 0)
        wout(j0 + 1, 1)
        return carry

    lax.fori_loop(0, (OUTER - 1) // 2, pair, 0)
    drain(0)
    wout(OUTER - 1, 0)


def _s1(y3, src, chunk):
    run = functools.partial(
        pl.kernel, mesh=_mesh(),
        out_type=jax.ShapeDtypeStruct((EC, W), jnp.float32),
        scratch_types=[pltpu.VMEM((PWG, CHG), jnp.int32),
                       pltpu.VMEM((2, GB, W), jnp.float32),
                       pltpu.SemaphoreType.DMA,
                       pltpu.SemaphoreType.DMA],
        compiler_params=pltpu.CompilerParams(use_tc_tiling_on_sc=True),
    )(functools.partial(_s1_body, chunk))
    return run(y3, src)


def _k2_body(g_ref, ee_ref, ea_ref, wr2_ref, tcat_ref, c_ref, msg_ref):
    B = g_ref.shape[0]
    gf = g_ref[...].astype(jnp.float32)
    acc = jnp.dot(ee_ref[...], wr2_ref[...], preferred_element_type=jnp.float32)
    idx = ea_ref[...]
    cols = lax.broadcasted_iota(jnp.int32, (B, 12), 1)
    tgt = cols % 3
    kk = cols // 3
    sel = jnp.where(kk == 0, idx[:, 0:1],
                    jnp.where(kk == 1, idx[:, 1:2],
                              jnp.where(kk == 2, idx[:, 2:3], idx[:, 3:4])))
    oh = (sel == tgt).astype(jnp.float32)
    acc = acc + jnp.dot(oh, tcat_ref[...], preferred_element_type=jnp.float32)
    v = gf + acc
    msg_ref[...] = _gelu(v) * c_ref[...]


def _k2(g_f, ee, ea, wr2s, tcat, cvec, chunk):
    B = 2560
    off = chunk * (EC // B)
    return pl.pallas_call(
        _k2_body,
        grid=(EC // B,),
        in_specs=[pl.BlockSpec((B, W), lambda i: (i, 0)),
                  pl.BlockSpec((B, H), lambda i: (i + off, 0)),
                  pl.BlockSpec((B, 4), lambda i: (i + off, 0)),
                  pl.BlockSpec((H, W), lambda i: (0, 0)),
                  pl.BlockSpec((12, W), lambda i: (0, 0)),
                  pl.BlockSpec((1, W), lambda i: (0, 0))],
        out_specs=pl.BlockSpec((B, W), lambda i: (i, 0)),
        out_shape=jax.ShapeDtypeStruct((EC, W), jnp.float32),
    )(g_f, ee, ea, wr2s, tcat, cvec)


def _s2_body(chunk, msg_hbm, dst_hbm, agg_hbm, idx_v, mbuf, zbuf, sem0, sem1,
             agg_s):
    cid = lax.axis_index("c")
    sid = lax.axis_index("s")
    wid = sid * NC + cid
    row0 = wid * PW
    sems = (sem0, sem1)

    for r in range(16):
        for l in range(8):
            zbuf[r, pl.ds(l * 16, 16)] = jnp.zeros((16,), jnp.float32)
    stripe0 = sid * 624
    nz = jnp.where(sid == NS - 1, 40, 39)

    def zstep(t, carry):
        pltpu.sync_copy(zbuf, agg_s.at[pl.ds(stripe0 + t * 16, 16)])
        return carry

    lax.fori_loop(0, nz, zstep, 0)
    plsc.subcore_barrier()

    pltpu.sync_copy(dst_hbm.at[chunk, wid], idx_v)

    def load(j, b):
        pltpu.async_copy(msg_hbm.at[pl.ds((row0 + j) * CH, CH)],
                         mbuf.at[b], sems[b])

    def wait(b):
        pltpu.make_async_copy(msg_hbm.at[pl.ds(row0 * CH, CH)],
                              mbuf.at[b], sems[b]).wait()

    def scat(j, b):
        pltpu.sync_copy(mbuf.at[b], agg_s.at[idx_v.at[j]], add=True)

    load(0, 0)

    def pair(p, carry):
        j0 = 2 * p
        wait(0)
        load(j0 + 1, 1)
        scat(j0, 0)
        wait(1)
        load(j0 + 2, 0)
        scat(j0 + 1, 1)
        return carry

    lax.fori_loop(0, (PW - 1) // 2, pair, 0)
    wait(0)
    scat(PW - 1, 0)
    plsc.subcore_barrier()

    def dstep(t, carry):
        pltpu.sync_copy(agg_s.at[pl.ds(stripe0 + t * 16, 16)], zbuf)
        pltpu.sync_copy(zbuf, agg_hbm.at[cid, pl.ds(stripe0 + t * 16, 16)])
        return carry

    lax.fori_loop(0, nz, dstep, 0)


def _s2(msg, dst, chunk):
    run = functools.partial(
        pl.kernel, mesh=_mesh(),
        out_type=jax.ShapeDtypeStruct((NC, N, W), jnp.float32),
        scratch_types=[pltpu.VMEM((PW, CH), jnp.int32),
                       pltpu.VMEM((2, CH, W), jnp.float32),
                       pltpu.VMEM((16, W), jnp.float32),
                       pltpu.SemaphoreType.DMA,
                       pltpu.SemaphoreType.DMA,
                       pltpu.VMEM_SHARED((N, W), jnp.float32)],
        compiler_params=pltpu.CompilerParams(use_tc_tiling_on_sc=True),
    )(functools.partial(_s2_body, chunk))
    return run(msg, dst)


def _k3_body(x_ref, z_ref, *rest):
    part_refs = rest[:2 * C]
    wp_ref, bp_ref, o0_ref, o1_ref = rest[2 * C:]
    xx2 = z_ref[...]
    for p in part_refs:
        xx2 = xx2 + p[...]
    out = jnp.dot(xx2, wp_ref[...], preferred_element_type=jnp.float32)
    out = out + bp_ref[...]
    o1_ref[...] = out
    o0_ref[...] = x_ref[...] + out


def _k3(x, z, parts, w_post, b_post):
    R = 1000
    blk = pl.BlockSpec((R, W), lambda i: (i, 0))
    return pl.pallas_call(
        _k3_body,
        grid=(N // R,),
        in_specs=[blk, blk] + [blk] * (2 * C)
                 + [pl.BlockSpec((W, W), lambda i: (0, 0)),
                    pl.BlockSpec((1, W), lambda i: (0, 0))],
        out_specs=[blk, blk],
        out_shape=[jax.ShapeDtypeStruct((N, W), jnp.float32),
                   jax.ShapeDtypeStruct((N, W), jnp.float32)],
    )(x, z, *parts, w_post, b_post)


def kernel(x, edge_index, edge_attr, edge_embed,
           W_pre, b_pre,
           W_msg0, b_msg0, W_msg1, b_msg1, W_msg2, b_msg2, W_msg3, b_msg3,
           W_r0, b_r0, W_r1, b_r1, W_r2, b_r2,
           W_post, b_post, init0,
           emb0, emb1, emb2, emb3, init0_e):
    s = jnp.exp(init0[-1])
    c = jnp.exp(init0[0])
    ex = jnp.exp(init0_e)
    xw = ex / jnp.sqrt(jnp.sum(ex))

    wr2s = W_r2 * (s * 0.5)
    tsrc = jnp.concatenate([emb0[0:3] * xw[0], emb1[0:3] * xw[1],
                            emb2[0:3] * xw[2], emb3[0:3] * xw[3]], axis=0)
    tcat = tsrc @ wr2s
    bias3 = (b_msg3 + s * b_r2).reshape(1, W)
    cvec = jnp.full((1, W), c, jnp.float32)

    src = edge_index[0, 0].reshape(C, NW, PWG, CHG)
    dst = edge_index[0, 1].reshape(C, NW, PW, CH)
    ea = edge_attr[0]
    ee = edge_embed[0]

    y3, z = _k1(x, W_pre, b_pre.reshape(1, W), W_msg3, bias3,
                W_msg0, b_msg0.reshape(1, W))

    parts = []
    for i in range(C):
        g_f = _s1(y3, src, i)
        msg = _k2(g_f, ee, ea, wr2s, tcat, cvec, i)
        agg2 = _s2(msg, dst, i)
        parts.append(agg2[0])
        parts.append(agg2[1])

    o0, o1 = _k3(x, z, parts, W_post, b_post.reshape(1, W))
    return (o0, o1, edge_embed)


# speedup vs baseline: 1.0154x; 1.0154x over previous
import functools

import jax
import jax.numpy as jnp
from jax import lax
from jax.experimental import pallas as pl
from jax.experimental.pallas import tpu as pltpu
from jax.experimental.pallas import tpu_sc as plsc

def _gelu(t):
    return 0.5 * t * (1.0 + lax.erf(t * 0.7071067811865476))


N = 10000
E = 320000
W = 128
H = 64

NC = 2
NS = 16
NW = NC * NS
C = 5
EC = E // C
CH = 80
PW = EC // (NW * CH)
CHG = 80
K_FIRE = 5
PWG = EC // (NW * CHG)
OUTER = PWG // K_FIRE
GB = K_FIRE * CHG

def _mesh():
    return plsc.VectorSubcoreMesh(core_axis_name="c", subcore_axis_name="s",
                                  num_cores=NC, num_subcores=NS)


def _k1_body(x_ref, wpre_ref, bpre_ref, w3_ref, b3_ref, w0_ref, b0_ref,
             y3_ref, z_ref):
    xx = jnp.dot(x_ref[...], wpre_ref[...], preferred_element_type=jnp.float32)
    xx = xx + bpre_ref[...]
    m = jnp.mean(xx, axis=-1, keepdims=True)
    v = jnp.mean(jnp.square(xx - m), axis=-1, keepdims=True)
    xx = (xx - m) / jnp.sqrt(v + 1e-5)
    y3 = jnp.dot(xx, w3_ref[...], preferred_element_type=jnp.float32)
    y3_ref[...] = y3 + b3_ref[...]
    z = jnp.dot(xx, w0_ref[...], preferred_element_type=jnp.float32)
    z_ref[...] = _gelu(z + b0_ref[...])


def _k1(x, w_pre, b_pre, w3, b3, w0, b0):
    R = 1000
    full = pl.BlockSpec((W, W), lambda i: (0, 0))
    row = pl.BlockSpec((1, W), lambda i: (0, 0))
    return pl.pallas_call(
        _k1_body,
        grid=(N // R,),
        in_specs=[pl.BlockSpec((R, W), lambda i: (i, 0)),
                  full, row, full, row, full, row],
        out_specs=[pl.BlockSpec((R, W), lambda i: (i, 0)),
                   pl.BlockSpec((R, W), lambda i: (i, 0))],
        out_shape=[jax.ShapeDtypeStruct((N, W), jnp.float32),
                   jax.ShapeDtypeStruct((N, W), jnp.float32)],
    )(x, w_pre, b_pre, w3, b3, w0, b0)


def _s1_body(chunk, y3_hbm, src_hbm, g_hbm, idx_v, gbuf, sem0, sem1):
    cid = lax.axis_index("c")
    sid = lax.axis_index("s")
    wid = sid * NC + cid
    row0 = wid * PWG
    pltpu.sync_copy(src_hbm.at[chunk, wid], idx_v)
    sems = (sem0, sem1)

    def fire(j, b):
        for u in range(K_FIRE):
            pltpu.async_copy(y3_hbm.at[idx_v.at[j * K_FIRE + u]],
                             gbuf.at[b, pl.ds(u * CHG, CHG)], sems[b])

    def drain(b):
        for u in range(K_FIRE):
            pltpu.make_async_copy(y3_hbm.at[idx_v.at[0]],
                                  gbuf.at[b, pl.ds(u * CHG, CHG)],
                                  sems[b]).wait()

    def wout(j, b):
        eoff = (row0 + j * K_FIRE) * CHG
        pltpu.sync_copy(gbuf.at[b], g_hbm.at[pl.ds(eoff, GB)])

    fire(0, 0)

    def pair(p, carry):
        j0 = 2 * p
        drain(0)
        fire(j0 + 1, 1)
        wout(j0, 0)
        drain(1)
        fire(j0 + 2, 0)
        wout(j0 + 1, 1)
        return carry

    lax.fori_loop(0, (OUTER - 1) // 2, pair, 0)
    drain(0)
    wout(OUTER - 1, 0)


def _s1(y3, src, chunk):
    run = functools.partial(
        pl.kernel, mesh=_mesh(),
        out_type=jax.ShapeDtypeStruct((EC, W), jnp.float32),
        scratch_types=[pltpu.VMEM((PWG, CHG), jnp.int32),
                       pltpu.VMEM((2, GB, W), jnp.float32),
                       pltpu.SemaphoreType.DMA,
                       pltpu.SemaphoreType.DMA],
        compiler_params=pltpu.CompilerParams(use_tc_tiling_on_sc=True),
    )(functools.partial(_s1_body, chunk))
    return run(y3, src)


def _k2_body(g_ref, ee_ref, ea_ref, wr2_ref, tcat_ref, c_ref, msg_ref):
    B = g_ref.shape[0]
    gf = g_ref[...].astype(jnp.float32)
    acc = jnp.dot(ee_ref[...], wr2_ref[...], preferred_element_type=jnp.float32)
    idx = ea_ref[...]
    cols = lax.broadcasted_iota(jnp.int32, (B, 12), 1)
    tgt = cols % 3
    kk = cols // 3
    sel = jnp.where(kk == 0, idx[:, 0:1],
                    jnp.where(kk == 1, idx[:, 1:2],
                              jnp.where(kk == 2, idx[:, 2:3], idx[:, 3:4])))
    oh = (sel == tgt).astype(jnp.float32)
    acc = acc + jnp.dot(oh, tcat_ref[...], preferred_element_type=jnp.float32)
    v = gf + acc
    msg_ref[...] = _gelu(v) * c_ref[...]


def _k2(g_f, ee, ea, wr2s, tcat, cvec, chunk):
    B = 2560
    off = chunk * (EC // B)
    return pl.pallas_call(
        _k2_body,
        grid=(EC // B,),
        in_specs=[pl.BlockSpec((B, W), lambda i: (i, 0)),
                  pl.BlockSpec((B, H), lambda i: (i + off, 0)),
                  pl.BlockSpec((B, 4), lambda i: (i + off, 0)),
                  pl.BlockSpec((H, W), lambda i: (0, 0)),
                  pl.BlockSpec((12, W), lambda i: (0, 0)),
                  pl.BlockSpec((1, W), lambda i: (0, 0))],
        out_specs=pl.BlockSpec((B, W), lambda i: (i, 0)),
        out_shape=jax.ShapeDtypeStruct((EC, W), jnp.float32),
    )(g_f, ee, ea, wr2s, tcat, cvec)


def _s2_body(chunk, msg_hbm, dst_hbm, agg_hbm, idx_v, mbuf, zbuf, sem0, sem1,
             agg_s):
    cid = lax.axis_index("c")
    sid = lax.axis_index("s")
    wid = sid * NC + cid
    row0 = wid * PW
    sems = (sem0, sem1)

    for r in range(16):
        for l in range(8):
            zbuf[r, pl.ds(l * 16, 16)] = jnp.zeros((16,), jnp.float32)
    stripe0 = sid * 624
    nz = jnp.where(sid == NS - 1, 40, 39)

    def zfire(t, carry):
        pltpu.async_copy(zbuf, agg_s.at[pl.ds(stripe0 + t * 16, 16)], sem0)
        return carry

    def zwait(t, carry):
        pltpu.make_async_copy(zbuf, agg_s.at[pl.ds(stripe0, 16)], sem0).wait()
        return carry

    lax.fori_loop(0, nz, zfire, 0)
    lax.fori_loop(0, nz, zwait, 0)
    plsc.subcore_barrier()

    pltpu.sync_copy(dst_hbm.at[chunk, wid], idx_v)

    def load(j, b):
        pltpu.async_copy(msg_hbm.at[pl.ds((row0 + j) * CH, CH)],
                         mbuf.at[b], sems[b])

    def wait(b):
        pltpu.make_async_copy(msg_hbm.at[pl.ds(row0 * CH, CH)],
                              mbuf.at[b], sems[b]).wait()

    def scat(j, b):
        pltpu.sync_copy(mbuf.at[b], agg_s.at[idx_v.at[j]], add=True)

    load(0, 0)

    def pair(p, carry):
        j0 = 2 * p
        wait(0)
        load(j0 + 1, 1)
        scat(j0, 0)
        wait(1)
        load(j0 + 2, 0)
        scat(j0 + 1, 1)
        return carry

    lax.fori_loop(0, (PW - 1) // 2, pair, 0)
    wait(0)
    scat(PW - 1, 0)
    plsc.subcore_barrier()

    def dfire(t, carry):
        pltpu.async_copy(agg_s.at[pl.ds(stripe0 + t * 16, 16)],
                         agg_hbm.at[cid, pl.ds(stripe0 + t * 16, 16)], sem0)
        return carry

    def dwait(t, carry):
        pltpu.make_async_copy(agg_s.at[pl.ds(stripe0, 16)],
                              agg_hbm.at[cid, pl.ds(stripe0, 16)], sem0).wait()
        return carry

    lax.fori_loop(0, nz, dfire, 0)
    lax.fori_loop(0, nz, dwait, 0)


def _s2(msg, dst, chunk):
    run = functools.partial(
        pl.kernel, mesh=_mesh(),
        out_type=jax.ShapeDtypeStruct((NC, N, W), jnp.float32),
        scratch_types=[pltpu.VMEM((PW, CH), jnp.int32),
                       pltpu.VMEM((2, CH, W), jnp.float32),
                       pltpu.VMEM((16, W), jnp.float32),
                       pltpu.SemaphoreType.DMA,
                       pltpu.SemaphoreType.DMA,
                       pltpu.VMEM_SHARED((N, W), jnp.float32)],
        compiler_params=pltpu.CompilerParams(use_tc_tiling_on_sc=True),
    )(functools.partial(_s2_body, chunk))
    return run(msg, dst)


def _k3_body(x_ref, z_ref, *rest):
    part_refs = rest[:2 * C]
    wp_ref, bp_ref, o0_ref, o1_ref = rest[2 * C:]
    xx2 = z_ref[...]
    for p in part_refs:
        xx2 = xx2 + p[...]
    out = jnp.dot(xx2, wp_ref[...], preferred_element_type=jnp.float32)
    out = out + bp_ref[...]
    o1_ref[...] = out
    o0_ref[...] = x_ref[...] + out


def _k3(x, z, parts, w_post, b_post):
    R = 1000
    blk = pl.BlockSpec((R, W), lambda i: (i, 0))
    return pl.pallas_call(
        _k3_body,
        grid=(N // R,),
        in_specs=[blk, blk] + [blk] * (2 * C)
                 + [pl.BlockSpec((W, W), lambda i: (0, 0)),
                    pl.BlockSpec((1, W), lambda i: (0, 0))],
        out_specs=[blk, blk],
        out_shape=[jax.ShapeDtypeStruct((N, W), jnp.float32),
                   jax.ShapeDtypeStruct((N, W), jnp.float32)],
    )(x, z, *parts, w_post, b_post)


def kernel(x, edge_index, edge_attr, edge_embed,
           W_pre, b_pre,
           W_msg0, b_msg0, W_msg1, b_msg1, W_msg2, b_msg2, W_msg3, b_msg3,
           W_r0, b_r0, W_r1, b_r1, W_r2, b_r2,
           W_post, b_post, init0,
           emb0, emb1, emb2, emb3, init0_e):
    s = jnp.exp(init0[-1])
    c = jnp.exp(init0[0])
    ex = jnp.exp(init0_e)
    xw = ex / jnp.sqrt(jnp.sum(ex))

    wr2s = W_r2 * (s * 0.5)
    tsrc = jnp.concatenate([emb0[0:3] * xw[0], emb1[0:3] * xw[1],
                            emb2[0:3] * xw[2], emb3[0:3] * xw[3]], axis=0)
    tcat = tsrc @ wr2s
    bias3 = (b_msg3 + s * b_r2).reshape(1, W)
    cvec = jnp.full((1, W), c, jnp.float32)

    src = edge_index[0, 0].reshape(C, NW, PWG, CHG)
    dst = edge_index[0, 1].reshape(C, NW, PW, CH)
    ea = edge_attr[0]
    ee = edge_embed[0]

    y3, z = _k1(x, W_pre, b_pre.reshape(1, W), W_msg3, bias3,
                W_msg0, b_msg0.reshape(1, W))

    parts = []
    for i in range(C):
        g_f = _s1(y3, src, i)
        msg = _k2(g_f, ee, ea, wr2s, tcat, cvec, i)
        agg2 = _s2(msg, dst, i)
        parts.append(agg2[0])
        parts.append(agg2[1])

    o0, o1 = _k3(x, z, parts, W_post, b_post.reshape(1, W))
    return (o0, o1, edge_embed)
